# baseline (speedup 1.0000x reference)
import functools

import jax
import jax.numpy as jnp
from jax import lax
from jax.experimental import pallas as pl
from jax.experimental.pallas import tpu as pltpu
from jax.experimental.pallas import tpu_sc as plsc

MAX_DELTA = 256
NUM_ROWS = 2 * MAX_DELTA + 1
D_MODEL = 128
LANES = 16

NUM_CORES = 2
NUM_SUBCORES = 16
NUM_WORKERS = NUM_CORES * NUM_SUBCORES

CHUNK = 128


def _make_sc_gather(total: int):
    assert total % (NUM_WORKERS * CHUNK) == 0
    per_worker = total // NUM_WORKERS
    n_chunks = per_worker // CHUNK
    mesh = plsc.VectorSubcoreMesh(core_axis_name="c", subcore_axis_name="s")

    @functools.partial(
        pl.kernel,
        out_type=jax.ShapeDtypeStruct((total, D_MODEL), jnp.float32),
        mesh=mesh,
        scratch_types=[
            pltpu.VMEM((CHUNK,), jnp.int32),
            pltpu.VMEM((CHUNK, D_MODEL), jnp.float32),
            pltpu.SemaphoreType.DMA,
        ],
    )
    def sc_gather(delta_hbm, table_hbm, out_hbm, idx_v, rows_v, sem):
        wid = lax.axis_index("s") * NUM_CORES + lax.axis_index("c")
        base = wid * per_worker

        def chunk_body(c, carry):
            off = base + c * CHUNK
            pltpu.sync_copy(delta_hbm.at[pl.ds(off, CHUNK)], idx_v)
            for j in range(CHUNK // LANES):
                sl = pl.ds(j * LANES, LANES)
                idx_v[sl] = jnp.clip(idx_v[sl] + MAX_DELTA, 0, 2 * MAX_DELTA)
            pltpu.async_copy(table_hbm.at[idx_v], rows_v, sem).wait()
            pltpu.sync_copy(rows_v, out_hbm.at[pl.ds(off, CHUNK)])
            return carry

        lax.fori_loop(0, n_chunks, chunk_body, 0)

    return sc_gather


def kernel(delta, table):
    total = delta.size
    flat = delta.reshape(total)
    out = _make_sc_gather(total)(flat, table)
    return out.reshape(*delta.shape, D_MODEL)
